# Initial kernel scaffold; baseline (speedup 1.0000x reference)
import jax
import jax.numpy as jnp
from jax.experimental import pallas as pl


def kernel(input, edge, w, a_src_dst):
    raise NotImplementedError("write your pallas kernel here")


# speedup vs baseline: 6.1615x; 6.1615x over previous
import functools

import jax
import jax.numpy as jnp
from jax import lax
from jax.experimental import pallas as pl
from jax.experimental.pallas import tpu as pltpu
from jax.experimental.pallas import tpu_sc as plsc

N_HEAD = 2
F = 128
FH = F // 2
NC = 2
NS = 16
L = 16
K = 128
AW = 144


def _s16_body(x_ref, w_ref, o_ref):
    o_ref[...] = lax.dot_general(
        x_ref[...], w_ref[...], (((1,), (1,)), ((), ())),
        preferred_element_type=jnp.float32)


def _sc_body(N, NP, EPT, NPT, FIN, xcat, dst2, srcp, stab, warr, out,
             accum, dstb, srcb, srs, srd, xrows, outbuf, wbuf, sem):
    c = lax.axis_index("c")
    s = lax.axis_index("s")
    nchunks = EPT // K
    zrows = NP // NS

    pltpu.sync_copy(warr.at[c], wbuf)

    def _zrow(r, _):
        for q in range(AW // L):
            outbuf[r, q * L:(q + 1) * L] = jnp.zeros((L,), jnp.float32)
        return _
    lax.fori_loop(0, K, _zrow, None)
    for t in range(zrows // K):
        pltpu.sync_copy(outbuf, accum.at[pl.ds(s * zrows + t * K, K)])
    plsc.subcore_barrier()

    iota = lax.iota(jnp.int32, L)
    zv = jnp.zeros((L,), jnp.float32)

    def _chunk(j, _):
        base = s * EPT + j * K
        pltpu.sync_copy(dst2.at[c, pl.ds(base, K)], dstb)
        pltpu.sync_copy(srcp.at[pl.ds(base, K)], srcb)
        cp1 = pltpu.async_copy(xcat.at[dstb], xrows, sem)
        cp2 = pltpu.async_copy(stab.at[srcb], srs, sem)
        cp3 = pltpu.async_copy(stab.at[dstb], srd, sem)
        cp1.wait()
        cp2.wait()
        cp3.wait()
        for g in range(K // L):
            g16 = g * L + iota
            ss0 = plsc.load_gather(srs, [g16, jnp.full((L,), 0, jnp.int32)])
            ss1 = plsc.load_gather(srs, [g16, jnp.full((L,), 1, jnp.int32)])
            sd0 = plsc.load_gather(srd, [g16, jnp.full((L,), 2, jnp.int32)])
            sd1 = plsc.load_gather(srd, [g16, jnp.full((L,), 3, jnp.int32)])
            t0 = ss0 + sd0
            t1 = ss1 + sd1
            ee0 = jnp.exp(-jnp.maximum(t0, 0.2 * t0))
            ee1 = jnp.exp(-jnp.maximum(t1, 0.2 * t1))
            for j16 in range(L):
                k = g * L + j16
                b0 = jnp.broadcast_to(ee0[j16], (L,))
                b1 = jnp.broadcast_to(ee1[j16], (L,))
                outbuf[k, F:F + L] = jnp.where(
                    iota == 0, b0, jnp.where(iota == 1, b1, zv))
                for q in range(FH // L):
                    xv = xrows[k, q * L:(q + 1) * L]
                    outbuf[k, q * L:(q + 1) * L] = xv * b0
                    outbuf[k, FH + q * L:FH + (q + 1) * L] = xv * b1
        pltpu.sync_copy(outbuf, accum.at[srcb], add=True)
        return _

    lax.fori_loop(0, nchunks, _chunk, None)
    plsc.subcore_barrier()

    def _mk_row(h):
        def _row(r, _):
            rv = outbuf[r, F:F + L]
            b = jnp.broadcast_to(rv[h], (L,))
            for q in range(FH // L):
                wv = wbuf[h * FH + q * L:h * FH + (q + 1) * L]
                xrows[r, q * L:(q + 1) * L] = (
                    outbuf[r, h * FH + q * L:h * FH + (q + 1) * L] * wv / b)
            return _
        return _row
    for t in range(NPT // FIN):
        r0 = s * NPT + t * FIN
        pltpu.sync_copy(accum.at[pl.ds(r0, FIN)], outbuf.at[pl.ds(0, FIN)])
        for h in range(N_HEAD):
            lax.fori_loop(0, FIN, _mk_row(h), None)
            pltpu.sync_copy(xrows.at[pl.ds(0, FIN)],
                            out.at[h, pl.ds(r0, FIN), pl.ds(c * FH, FH)])


def kernel(input, edge, w, a_src_dst):
    x = input.astype(jnp.float32)
    N = x.shape[0]
    E = edge.shape[1]
    src = edge[0, :].astype(jnp.int32)
    dst = edge[1, :].astype(jnp.int32)

    av = a_src_dst[:, :, 0].astype(jnp.float32)
    wv = w[:, 0, :].astype(jnp.float32)
    ws16 = jnp.zeros((16, F), jnp.float32)
    ws16 = ws16.at[0].set(wv[0] * av[0, :F])
    ws16 = ws16.at[1].set(wv[1] * av[1, :F])
    ws16 = ws16.at[2].set(wv[0] * av[0, F:])
    ws16 = ws16.at[3].set(wv[1] * av[1, F:])

    bn = 1000 if N % 1000 == 0 else (8 if N % 8 == 0 else N)
    s16 = pl.pallas_call(
        _s16_body,
        grid=(N // bn,),
        in_specs=[pl.BlockSpec((bn, F), lambda i: (i, 0)),
                  pl.BlockSpec((16, F), lambda i: (0, 0))],
        out_specs=pl.BlockSpec((bn, 16), lambda i: (i, 0)),
        out_shape=jax.ShapeDtypeStruct((N, 16), jnp.float32),
    )(x, ws16)

    NP = ((N + K) + 2 * K - 1) // (2 * K) * (2 * K)
    EPT = ((E + NS - 1) // NS + K - 1) // K * K
    EPAD = EPT * NS
    NPT = N // NS
    FIN = 125 if NPT % 125 == 0 else (NPT if NPT <= K else 1)

    stab = jnp.zeros((2 * NP, 16), jnp.float32)
    stab = stab.at[0:N].set(s16).at[NP:NP + N].set(s16)
    xcat = jnp.zeros((2 * NP, FH), jnp.float32)
    xcat = xcat.at[0:N].set(x[:, :FH]).at[NP:NP + N].set(x[:, FH:])
    dstp = jnp.full((EPAD,), N, jnp.int32).at[:E].set(dst)
    dst2 = jnp.stack([dstp, dstp + NP])
    srcp = jnp.full((EPAD,), N, jnp.int32).at[:E].set(src)
    warr = jnp.stack([jnp.concatenate([wv[0, :FH], wv[1, :FH]]),
                      jnp.concatenate([wv[0, FH:], wv[1, FH:]])])

    sc = functools.partial(
        pl.kernel,
        out_type=jax.ShapeDtypeStruct((N_HEAD, N, F), jnp.float32),
        mesh=plsc.VectorSubcoreMesh(core_axis_name="c", subcore_axis_name="s",
                                    num_cores=NC, num_subcores=NS),
        compiler_params=pltpu.CompilerParams(use_tc_tiling_on_sc=False,
                                             needs_layout_passes=False),
        scratch_types=[
            pltpu.VMEM_SHARED((NP, AW), jnp.float32),
            pltpu.VMEM((K,), jnp.int32),
            pltpu.VMEM((K,), jnp.int32),
            pltpu.VMEM((K, 16), jnp.float32),
            pltpu.VMEM((K, 16), jnp.float32),
            pltpu.VMEM((K, FH), jnp.float32),
            pltpu.VMEM((K, AW), jnp.float32),
            pltpu.VMEM((F,), jnp.float32),
            pltpu.SemaphoreType.DMA,
        ],
    )(functools.partial(_sc_body, N, NP, EPT, NPT, FIN))

    return sc(xcat, dst2, srcp, stab, warr)


# speedup vs baseline: 6.2891x; 1.0207x over previous
import functools

import jax
import jax.numpy as jnp
from jax import lax
from jax.experimental import pallas as pl
from jax.experimental.pallas import tpu as pltpu
from jax.experimental.pallas import tpu_sc as plsc

N_HEAD = 2
F = 128
FH = F // 2
NC = 2
NS = 16
L = 16
K = 64
XW = 64
AW = 144


def _s16_body(x_ref, w_ref, o_ref):
    o_ref[...] = lax.dot_general(
        x_ref[...], w_ref[...], (((1,), (1,)), ((), ())),
        preferred_element_type=jnp.float32)


def _sc_body(N, NP, EPT, NPT, FIN, xcat, dst2, srcp, stab, warr, out,
             accum, dstb0, srcb0, srs0, xr0, ob0, srd0, scb0,
             dstb1, srcb1, srs1, xr1, ob1, srd1, scb1,
             hbuf, wbuf, gsem0, gsem1, ssem0, ssem1):
    c = lax.axis_index("c")
    s = lax.axis_index("s")
    nchunks = EPT // K
    zrows = NP // NS
    iota = lax.iota(jnp.int32, L)
    zv = jnp.zeros((L,), jnp.float32)
    rings = ((dstb0, srcb0, srs0, xr0, ob0, srd0, scb0, gsem0, ssem0),
             (dstb1, srcb1, srs1, xr1, ob1, srd1, scb1, gsem1, ssem1))

    pltpu.sync_copy(warr.at[c], wbuf)

    def _zrow(r, _):
        for q in range(AW // L):
            ob0[r, q * L:(q + 1) * L] = zv
            ob1[r, q * L:(q + 1) * L] = zv
        return _
    lax.fori_loop(0, K, _zrow, None)
    for t in range(zrows // (2 * K)):
        pltpu.sync_copy(ob0, accum.at[pl.ds(s * zrows + 2 * t * K, K)])
        pltpu.sync_copy(ob1, accum.at[pl.ds(s * zrows + (2 * t + 1) * K, K)])
    plsc.subcore_barrier()


    def issue(j, b):
        dstb, srcb, srs, xr, _, srd, _, gsem, _ = rings[b]
        base = s * EPT + j * K
        pltpu.sync_copy(dst2.at[c, pl.ds(base, K)], dstb)
        pltpu.sync_copy(srcp.at[pl.ds(base, K)], srcb)
        pltpu.async_copy(xcat.at[dstb], xr, gsem)
        pltpu.async_copy(stab.at[srcb], srs, gsem)
        pltpu.async_copy(stab.at[dstb], srd, gsem)

    def drain_gather(b):
        dstb, srcb, srs, xr, _, srd, _, gsem, _ = rings[b]
        pltpu.make_async_copy(xcat.at[dstb], xr, gsem).wait()
        pltpu.make_async_copy(stab.at[srcb], srs, gsem).wait()
        pltpu.make_async_copy(stab.at[dstb], srd, gsem).wait()

    def drain_scatter(b):
        _, _, _, _, ob, _, scb, _, ssem = rings[b]
        pltpu.make_async_copy(ob, accum.at[scb], ssem).wait()

    def step(j, b):
        dstb, srcb, srs, xr, ob, srd, scb, gsem, ssem = rings[b]
        drain_gather(b)
        drain_scatter(b)
        pltpu.sync_copy(srcp.at[pl.ds(s * EPT + j * K, K)], scb)
        for g in range(K // L):
            g16 = g * L + iota
            ss0 = plsc.load_gather(srs, [g16, jnp.full((L,), 0, jnp.int32)])
            ss1 = plsc.load_gather(srs, [g16, jnp.full((L,), 1, jnp.int32)])
            sd0 = plsc.load_gather(srd, [g16, jnp.full((L,), 2, jnp.int32)])
            sd1 = plsc.load_gather(srd, [g16, jnp.full((L,), 3, jnp.int32)])
            t0 = ss0 + sd0
            t1 = ss1 + sd1
            ee0 = jnp.exp(-jnp.maximum(t0, 0.2 * t0))
            ee1 = jnp.exp(-jnp.maximum(t1, 0.2 * t1))
            for j16 in range(L):
                k = g * L + j16
                b0 = jnp.broadcast_to(ee0[j16], (L,))
                b1 = jnp.broadcast_to(ee1[j16], (L,))
                ob[k, F:F + L] = jnp.where(
                    iota == 0, b0, jnp.where(iota == 1, b1, zv))
                for q in range(FH // L):
                    xv = xr[k, q * L:(q + 1) * L]
                    ob[k, q * L:(q + 1) * L] = xv * b0
                    ob[k, FH + q * L:FH + (q + 1) * L] = xv * b1
        pltpu.async_copy(ob, accum.at[scb], ssem, add=True)
        issue(j + 2, b)

    pltpu.sync_copy(srcp.at[pl.ds(0, K)], scb0)
    pltpu.sync_copy(srcp.at[pl.ds(0, K)], scb1)
    pltpu.async_copy(ob0, accum.at[scb0], ssem0, add=True)
    pltpu.async_copy(ob1, accum.at[scb1], ssem1, add=True)
    issue(0, 0)
    issue(1, 1)

    def _pair(p, _):
        step(2 * p, 0)
        step(2 * p + 1, 1)
        return _
    lax.fori_loop(0, nchunks // 2, _pair, None)
    for b in range(2):
        drain_gather(b)
        drain_scatter(b)
    plsc.subcore_barrier()

    def _mk_row(h):
        def _row(r, _):
            rv = ob0[r, F:F + L]
            b = jnp.broadcast_to(rv[h], (L,))
            for q in range(FH // L):
                wv = wbuf[h * FH + q * L:h * FH + (q + 1) * L]
                hbuf[r, q * L:(q + 1) * L] = (
                    ob0[r, h * FH + q * L:h * FH + (q + 1) * L] * wv / b)
            return _
        return _row
    for t in range(NPT // FIN):
        r0 = s * NPT + t * FIN
        pltpu.sync_copy(accum.at[pl.ds(r0, FIN)], ob0.at[pl.ds(0, FIN)])
        for h in range(N_HEAD):
            lax.fori_loop(0, FIN, _mk_row(h), None)
            pltpu.sync_copy(hbuf,
                            out.at[h, pl.ds(r0, FIN), pl.ds(c * FH, FH)])


def kernel(input, edge, w, a_src_dst):
    x = input.astype(jnp.float32)
    N = x.shape[0]
    E = edge.shape[1]
    src = edge[0, :].astype(jnp.int32)
    dst = edge[1, :].astype(jnp.int32)

    av = a_src_dst[:, :, 0].astype(jnp.float32)
    wv = w[:, 0, :].astype(jnp.float32)
    ws16 = jnp.zeros((16, F), jnp.float32)
    ws16 = ws16.at[0].set(wv[0] * av[0, :F])
    ws16 = ws16.at[1].set(wv[1] * av[1, :F])
    ws16 = ws16.at[2].set(wv[0] * av[0, F:])
    ws16 = ws16.at[3].set(wv[1] * av[1, F:])

    bn = 1000 if N % 1000 == 0 else (8 if N % 8 == 0 else N)
    s16 = pl.pallas_call(
        _s16_body,
        grid=(N // bn,),
        in_specs=[pl.BlockSpec((bn, F), lambda i: (i, 0)),
                  pl.BlockSpec((16, F), lambda i: (0, 0))],
        out_specs=pl.BlockSpec((bn, 16), lambda i: (i, 0)),
        out_shape=jax.ShapeDtypeStruct((N, 16), jnp.float32),
    )(x, ws16)

    NP = (N + 1 + 2 * K * NS - 1) // (2 * K * NS) * (2 * K * NS)
    EPT = ((E + NS - 1) // NS + 2 * K - 1) // (2 * K) * (2 * K)
    EPAD = EPT * NS + 2 * K
    NPT = N // NS
    FIN = 25 if NPT % 25 == 0 else (NPT if NPT <= K else 1)

    stab = jnp.zeros((2 * NP, 16), jnp.float32)
    stab = stab.at[0:N].set(s16).at[NP:NP + N].set(s16)
    xcat = jnp.zeros((2 * NP, XW), jnp.float32)
    xcat = xcat.at[0:N, :FH].set(x[:, :FH]).at[NP:NP + N, :FH].set(x[:, FH:])
    dstp = jnp.full((EPAD,), N, jnp.int32).at[:E].set(dst)
    dst2 = jnp.stack([dstp, dstp + NP])
    srcp = jnp.full((EPAD,), N, jnp.int32).at[:E].set(src)
    warr = jnp.stack([jnp.concatenate([wv[0, :FH], wv[1, :FH]]),
                      jnp.concatenate([wv[0, FH:], wv[1, FH:]])])

    ring = lambda: [
        pltpu.VMEM((K,), jnp.int32),
        pltpu.VMEM((K,), jnp.int32),
        pltpu.VMEM((K, 16), jnp.float32),
        pltpu.VMEM((K, XW), jnp.float32),
        pltpu.VMEM((K, AW), jnp.float32),
        pltpu.VMEM((K, 16), jnp.float32),
        pltpu.VMEM((K,), jnp.int32),
    ]
    sc = functools.partial(
        pl.kernel,
        out_type=jax.ShapeDtypeStruct((N_HEAD, N, F), jnp.float32),
        mesh=plsc.VectorSubcoreMesh(core_axis_name="c", subcore_axis_name="s",
                                    num_cores=NC, num_subcores=NS),
        compiler_params=pltpu.CompilerParams(use_tc_tiling_on_sc=False,
                                             needs_layout_passes=False),
        scratch_types=[pltpu.VMEM_SHARED((NP, AW), jnp.float32)]
        + ring() + ring()
        + [pltpu.VMEM((FIN, FH), jnp.float32),
           pltpu.VMEM((F,), jnp.float32),
           pltpu.SemaphoreType.DMA,
           pltpu.SemaphoreType.DMA,
           pltpu.SemaphoreType.DMA,
           pltpu.SemaphoreType.DMA],
    )(functools.partial(_sc_body, N, NP, EPT, NPT, FIN))

    return sc(xcat, dst2, srcp, stab, warr)
